# baseline (speedup 1.0000x reference)
import functools

import jax
import jax.numpy as jnp
from jax import lax
from jax.experimental import pallas as pl
from jax.experimental.pallas import tpu as pltpu
from jax.experimental.pallas import tpu_sc as plsc

N_MID = 1000000
DIM = 64
SEQ = 50
BATCH = 4096
ROWS = BATCH * SEQ

NC = 2
NS = 16
NW = NC * NS
LANES = 16

RSTRIDE = 65
SLAB = 256
N_SLABS_MAIN = 3904
SLABS_PER_W = N_SLABS_MAIN // NW
N_SLABS_FULL = N_MID // SLAB
TAIL_V0 = N_SLABS_FULL * SLAB
TAIL_W = N_MID - TAIL_V0

IDX_MINOR = 128
IDX_ROWS = ROWS // IDX_MINOR
IDXR_PER_W = IDX_ROWS // NW
IDXR_PER_CHUNK = 5
N_CHUNKS = IDXR_PER_W // IDXR_PER_CHUNK
CHUNK = IDXR_PER_CHUNK * IDX_MINOR

PAIR_ROWS = ROWS // 2
SEQ_PAIRS = SEQ // 2
TC_BLOCK_SEQS = 16
TC_BLOCK_ROWS = TC_BLOCK_SEQS * SEQ_PAIRS


def _transpose_slab(src_ref, dst_ref, iota_r, width):
    def c_body(c, carry):
        bvec = iota_r + c * (LANES * RSTRIDE)
        for d in range(DIM):
            v = src_ref[d, pl.ds(c * LANES, LANES)]
            plsc.store_scatter(dst_ref, [bvec + d], v)
        return carry

    lax.fori_loop(0, width // LANES, c_body, 0)


def _sc_transpose(table_t_hbm, tail_hbm, out_hbm, src0, src1, dst0, dst1,
                  si0, si1, so0, so1):
    wid = lax.axis_index("s") * NC + lax.axis_index("c")
    iota_r = lax.iota(jnp.int32, LANES) * RSTRIDE
    si = (si0, si1)
    so = (so0, so1)
    src = (src0, src1)
    dst = (dst0, dst1)

    def fire_in(p, slab, sem):
        v0 = slab * SLAB
        return pltpu.async_copy(
            table_t_hbm.at[:, pl.ds(v0, SLAB)], src[p], sem)

    fire_in(0, wid, si[0]).wait()

    def outer(k2, carry):
        for p in range(2):
            j = k2 * 2 + p
            slab = j * NW + wid
            nxt = j + 1

            @pl.when(nxt < SLABS_PER_W)
            def _():
                fire_in(1 - p, nxt * NW + wid, si[1 - p])

            @pl.when(j >= 2)
            def _():
                pltpu.make_async_copy(
                    dst[p], out_hbm.at[pl.ds(0, SLAB * RSTRIDE)], so[p]
                ).wait()

            _transpose_slab(src[p], dst[p], iota_r, SLAB)
            pltpu.async_copy(
                dst[p],
                out_hbm.at[pl.ds(slab * SLAB * RSTRIDE, SLAB * RSTRIDE)],
                so[p],
            )
            @pl.when(nxt < SLABS_PER_W)
            def _():
                pltpu.make_async_copy(
                    table_t_hbm.at[:, pl.ds(0, SLAB)], src[1 - p],
                    si[1 - p],
                ).wait()
        return carry

    lax.fori_loop(0, SLABS_PER_W // 2, outer, 0)
    for p in range(2):
        pltpu.make_async_copy(
            dst[p], out_hbm.at[pl.ds(0, SLAB * RSTRIDE)], so[p]
        ).wait()

    for w, slab in ((0, N_SLABS_MAIN), (1, N_SLABS_MAIN + 1)):

        @pl.when(wid == w)
        def _():
            fire_in(0, slab, si[0]).wait()
            _transpose_slab(src0, dst0, iota_r, SLAB)
            pltpu.async_copy(
                dst0,
                out_hbm.at[pl.ds(slab * SLAB * RSTRIDE, SLAB * RSTRIDE)],
                so[0],
            ).wait()

    @pl.when(wid == 2)
    def _():
        pltpu.async_copy(
            tail_hbm,
            src0.at[:, pl.ds(0, 2 * TAIL_W)],
            si[0],
        ).wait()
        _transpose_slab(
            src0,
            dst0.at[pl.ds(0, TAIL_W * RSTRIDE)],
            iota_r,
            TAIL_W,
        )
        pltpu.async_copy(
            dst0.at[pl.ds(0, TAIL_W * RSTRIDE)],
            out_hbm.at[pl.ds(TAIL_V0 * RSTRIDE, TAIL_W * RSTRIDE)],
            so[0],
        ).wait()


def _sc_gather(idx_hbm, table_hbm, out_hbm, idx_v, rows_v, gsem, ssem0, ssem1):
    wid = lax.axis_index("s") * NC + lax.axis_index("c")
    idxr0 = wid * IDXR_PER_W
    row0 = wid * IDXR_PER_W * IDX_MINOR

    ssems = (ssem0, ssem1)
    store_handles = [None, None]
    for k in range(N_CHUNKS):
        p = k % 2
        if store_handles[p] is not None:
            store_handles[p].wait()
        pltpu.sync_copy(
            idx_hbm.at[pl.ds(idxr0 + k * IDXR_PER_CHUNK, IDXR_PER_CHUNK)],
            idx_v.at[p],
        )
        gathers = []
        for j in range(IDXR_PER_CHUNK):
            gathers.append(
                pltpu.async_copy(
                    table_hbm.at[idx_v.at[p, j]],
                    rows_v.at[p, pl.ds(j * IDX_MINOR, IDX_MINOR)],
                    gsem,
                )
            )
        for g in gathers:
            g.wait()
        store_handles[p] = pltpu.async_copy(
            rows_v.at[p, :, pl.ds(0, DIM)],
            out_hbm.at[pl.ds(row0 + k * CHUNK, CHUNK)],
            ssems[p],
        )
    for h in store_handles:
        if h is not None:
            h.wait()


def _tc_add(rows_ref, pos_ref, out_ref):
    for b in range(TC_BLOCK_SEQS):
        sl = pl.ds(b * SEQ_PAIRS, SEQ_PAIRS)
        out_ref[sl, :] = rows_ref[sl, :] + pos_ref[0]


def kernel(item, nbr_mask, i_ids, item_input_lookup, position_embedding):
    del nbr_mask, i_ids

    mesh = plsc.VectorSubcoreMesh(core_axis_name="c", subcore_axis_name="s")

    table_t = item_input_lookup.T
    tail_pad = jnp.pad(table_t[:, TAIL_V0:], ((0, 0), (0, 128 - TAIL_W)))
    transpose = functools.partial(
        pl.kernel,
        mesh=mesh,
        out_type=jax.ShapeDtypeStruct((N_MID * RSTRIDE,), jnp.float32),
        scratch_types=[
            pltpu.VMEM((DIM, SLAB), jnp.float32),
            pltpu.VMEM((DIM, SLAB), jnp.float32),
            pltpu.VMEM((SLAB * RSTRIDE,), jnp.float32),
            pltpu.VMEM((SLAB * RSTRIDE,), jnp.float32),
            pltpu.SemaphoreType.DMA,
            pltpu.SemaphoreType.DMA,
            pltpu.SemaphoreType.DMA,
            pltpu.SemaphoreType.DMA,
        ],
        compiler_params=pltpu.CompilerParams(
            use_tc_tiling_on_sc=True, needs_layout_passes=False
        ),
    )(_sc_transpose)
    dense_flat = transpose(table_t, tail_pad)
    dense2d = dense_flat.reshape(N_MID, RSTRIDE)

    idx2d = item.reshape(IDX_ROWS, IDX_MINOR)
    gather = functools.partial(
        pl.kernel,
        mesh=mesh,
        out_type=jax.ShapeDtypeStruct((ROWS, DIM), jnp.float32),
        scratch_types=[
            pltpu.VMEM((2, IDXR_PER_CHUNK, IDX_MINOR), jnp.int32),
            pltpu.VMEM((2, CHUNK, RSTRIDE), jnp.float32),
            pltpu.SemaphoreType.DMA,
            pltpu.SemaphoreType.DMA,
            pltpu.SemaphoreType.DMA,
        ],
        compiler_params=pltpu.CompilerParams(use_tc_tiling_on_sc=False),
    )(_sc_gather)
    gathered = gather(idx2d, dense2d)

    pairs = gathered.reshape(PAIR_ROWS, 2 * DIM)
    pos_pairs = position_embedding.reshape(1, SEQ_PAIRS, 2 * DIM)
    out_pairs = pl.pallas_call(
        _tc_add,
        grid=(PAIR_ROWS // TC_BLOCK_ROWS,),
        in_specs=[
            pl.BlockSpec((TC_BLOCK_ROWS, 2 * DIM), lambda i: (i, 0)),
            pl.BlockSpec((1, SEQ_PAIRS, 2 * DIM), lambda i: (0, 0, 0)),
        ],
        out_specs=pl.BlockSpec((TC_BLOCK_ROWS, 2 * DIM), lambda i: (i, 0)),
        out_shape=jax.ShapeDtypeStruct((PAIR_ROWS, 2 * DIM), jnp.float32),
    )(pairs, pos_pairs)
    return out_pairs.reshape(BATCH, SEQ, DIM)


# speedup vs baseline: 2.1193x; 2.1193x over previous
import functools

import jax
import jax.numpy as jnp
from jax import lax
from jax.experimental import pallas as pl
from jax.experimental.pallas import tpu as pltpu
from jax.experimental.pallas import tpu_sc as plsc

N_MID = 1000000
DIM = 64
SEQ = 50
BATCH = 4096
ROWS = BATCH * SEQ

NC = 2
NS = 16
NW = NC * NS
LANES = 16

SLAB = 256
SSTR = 69
N_SLABS_MAIN = 3904
SLABS_PER_W = N_SLABS_MAIN // NW
N_SLABS_FULL = N_MID // SLAB
TAIL_V0 = N_SLABS_FULL * SLAB
TAIL_W = N_MID - TAIL_V0

IDX_MINOR = 128
IDX_ROWS = ROWS // IDX_MINOR
IDXR_PER_W = IDX_ROWS // NW
IDXR_PER_CHUNK = 5
N_CHUNKS = IDXR_PER_W // IDXR_PER_CHUNK
CHUNK = IDXR_PER_CHUNK * IDX_MINOR

PAIR_ROWS = ROWS // 2
SEQ_PAIRS = SEQ // 2
TC_BLOCK_SEQS = 64
TC_BLOCK_ROWS = TC_BLOCK_SEQS * SEQ_PAIRS


def _transpose_slab(src_ref, stage_ref, dst_ref, iota_s, width):
    def c_body(c, carry):
        for d in range(DIM):
            v = src_ref[d, pl.ds(c * LANES, LANES)]
            plsc.store_scatter(stage_ref, [iota_s + d], v)
        for i in range(LANES):
            for g in range(DIM // LANES):
                r = stage_ref[pl.ds(i * SSTR + g * LANES, LANES)]
                dst_ref[pl.ds((c * LANES + i) * DIM + g * LANES, LANES)] = r
        return carry

    lax.fori_loop(0, width // LANES, c_body, 0)


def _sc_transpose(table_t_hbm, tail_hbm, out_hbm, src0, src1, dst0, dst1,
                  stage_v, si0, si1, so0, so1):
    wid = lax.axis_index("s") * NC + lax.axis_index("c")
    iota_s = lax.iota(jnp.int32, LANES) * SSTR
    si = (si0, si1)
    so = (so0, so1)
    src = (src0, src1)
    dst = (dst0, dst1)

    def fire_in(p, slab, sem):
        v0 = slab * SLAB
        return pltpu.async_copy(
            table_t_hbm.at[:, pl.ds(v0, SLAB)], src[p], sem)

    fire_in(0, wid, si[0]).wait()

    def outer(k2, carry):
        for p in range(2):
            j = k2 * 2 + p
            slab = j * NW + wid
            nxt = j + 1

            @pl.when(nxt < SLABS_PER_W)
            def _():
                fire_in(1 - p, nxt * NW + wid, si[1 - p])

            @pl.when(j >= 2)
            def _():
                pltpu.make_async_copy(
                    dst[p], out_hbm.at[pl.ds(0, SLAB * DIM)], so[p]
                ).wait()

            _transpose_slab(src[p], stage_v, dst[p], iota_s, SLAB)
            pltpu.async_copy(
                dst[p],
                out_hbm.at[pl.ds(slab * SLAB * DIM, SLAB * DIM)],
                so[p],
            )
            @pl.when(nxt < SLABS_PER_W)
            def _():
                pltpu.make_async_copy(
                    table_t_hbm.at[:, pl.ds(0, SLAB)], src[1 - p],
                    si[1 - p],
                ).wait()
        return carry

    lax.fori_loop(0, SLABS_PER_W // 2, outer, 0)
    for p in range(2):
        pltpu.make_async_copy(
            dst[p], out_hbm.at[pl.ds(0, SLAB * DIM)], so[p]
        ).wait()

    for w, slab in ((0, N_SLABS_MAIN), (1, N_SLABS_MAIN + 1)):

        @pl.when(wid == w)
        def _():
            fire_in(0, slab, si[0]).wait()
            _transpose_slab(src0, stage_v, dst0, iota_s, SLAB)
            pltpu.async_copy(
                dst0,
                out_hbm.at[pl.ds(slab * SLAB * DIM, SLAB * DIM)],
                so[0],
            ).wait()

    @pl.when(wid == 2)
    def _():
        pltpu.async_copy(
            tail_hbm,
            src0.at[:, pl.ds(0, 2 * TAIL_W)],
            si[0],
        ).wait()
        _transpose_slab(
            src0,
            stage_v,
            dst0.at[pl.ds(0, TAIL_W * DIM)],
            iota_s,
            TAIL_W,
        )
        pltpu.async_copy(
            dst0.at[pl.ds(0, TAIL_W * DIM)],
            out_hbm.at[pl.ds(TAIL_V0 * DIM, TAIL_W * DIM)],
            so[0],
        ).wait()


def _sc_gather(idx_hbm, table_hbm, out_hbm, idx_v, rows_v, gsem, ssem0, ssem1):
    wid = lax.axis_index("s") * NC + lax.axis_index("c")
    idxr0 = wid * IDXR_PER_W
    row0 = wid * IDXR_PER_W * IDX_MINOR

    ssems = (ssem0, ssem1)
    store_handles = [None, None]
    for k in range(N_CHUNKS):
        p = k % 2
        if store_handles[p] is not None:
            store_handles[p].wait()
        pltpu.sync_copy(
            idx_hbm.at[pl.ds(idxr0 + k * IDXR_PER_CHUNK, IDXR_PER_CHUNK)],
            idx_v.at[p],
        )
        gathers = []
        for j in range(IDXR_PER_CHUNK):
            gathers.append(
                pltpu.async_copy(
                    table_hbm.at[idx_v.at[p, j]],
                    rows_v.at[p, pl.ds(j * IDX_MINOR, IDX_MINOR)],
                    gsem,
                )
            )
        for g in gathers:
            g.wait()
        store_handles[p] = pltpu.async_copy(
            rows_v.at[p],
            out_hbm.at[pl.ds(row0 + k * CHUNK, CHUNK)],
            ssems[p],
        )
    for h in store_handles:
        if h is not None:
            h.wait()


def _tc_add(rows_ref, pos_ref, out_ref):
    for b in range(TC_BLOCK_SEQS):
        sl = pl.ds(b * SEQ_PAIRS, SEQ_PAIRS)
        out_ref[sl, :] = rows_ref[sl, :] + pos_ref[0]


def kernel(item, nbr_mask, i_ids, item_input_lookup, position_embedding):
    del nbr_mask, i_ids

    mesh = plsc.VectorSubcoreMesh(core_axis_name="c", subcore_axis_name="s")

    table_t = item_input_lookup.T
    tail_pad = jnp.pad(table_t[:, TAIL_V0:], ((0, 0), (0, 128 - TAIL_W)))
    transpose = functools.partial(
        pl.kernel,
        mesh=mesh,
        out_type=jax.ShapeDtypeStruct((N_MID * DIM,), jnp.float32),
        scratch_types=[
            pltpu.VMEM((DIM, SLAB), jnp.float32),
            pltpu.VMEM((DIM, SLAB), jnp.float32),
            pltpu.VMEM((SLAB * DIM,), jnp.float32),
            pltpu.VMEM((SLAB * DIM,), jnp.float32),
            pltpu.VMEM((LANES * SSTR,), jnp.float32),
            pltpu.SemaphoreType.DMA,
            pltpu.SemaphoreType.DMA,
            pltpu.SemaphoreType.DMA,
            pltpu.SemaphoreType.DMA,
        ],
        compiler_params=pltpu.CompilerParams(
            use_tc_tiling_on_sc=True, needs_layout_passes=False
        ),
    )(_sc_transpose)
    dense_flat = transpose(table_t, tail_pad)
    dense2d = dense_flat.reshape(N_MID, DIM)

    idx2d = item.reshape(IDX_ROWS, IDX_MINOR)
    gather = functools.partial(
        pl.kernel,
        mesh=mesh,
        out_type=jax.ShapeDtypeStruct((ROWS, DIM), jnp.float32),
        scratch_types=[
            pltpu.VMEM((2, IDXR_PER_CHUNK, IDX_MINOR), jnp.int32),
            pltpu.VMEM((2, CHUNK, DIM), jnp.float32),
            pltpu.SemaphoreType.DMA,
            pltpu.SemaphoreType.DMA,
            pltpu.SemaphoreType.DMA,
        ],
        compiler_params=pltpu.CompilerParams(use_tc_tiling_on_sc=False),
    )(_sc_gather)
    gathered = gather(idx2d, dense2d)

    pairs = gathered.reshape(PAIR_ROWS, 2 * DIM)
    pos_pairs = position_embedding.reshape(1, SEQ_PAIRS, 2 * DIM)
    out_pairs = pl.pallas_call(
        _tc_add,
        grid=(PAIR_ROWS // TC_BLOCK_ROWS,),
        in_specs=[
            pl.BlockSpec((TC_BLOCK_ROWS, 2 * DIM), lambda i: (i, 0)),
            pl.BlockSpec((1, SEQ_PAIRS, 2 * DIM), lambda i: (0, 0, 0)),
        ],
        out_specs=pl.BlockSpec((TC_BLOCK_ROWS, 2 * DIM), lambda i: (i, 0)),
        out_shape=jax.ShapeDtypeStruct((PAIR_ROWS, 2 * DIM), jnp.float32),
    )(pairs, pos_pairs)
    return out_pairs.reshape(BATCH, SEQ, DIM)


# speedup vs baseline: 3.0081x; 1.4194x over previous
import functools

import jax
import jax.numpy as jnp
from jax import lax
from jax.experimental import pallas as pl
from jax.experimental.pallas import tpu as pltpu
from jax.experimental.pallas import tpu_sc as plsc

N_MID = 1000000
DIM = 64
SEQ = 50
BATCH = 4096
ROWS = BATCH * SEQ

NC = 2
NS = 16
NW = NC * NS
LANES = 16

PDIM = 128

IDX_MINOR = 128
IDX_ROWS = ROWS // IDX_MINOR
IDXR_PER_W = IDX_ROWS // NW
IDXR_PER_CHUNK = 2
N_CHUNKS = IDXR_PER_W // IDXR_PER_CHUNK
CHUNK = IDXR_PER_CHUNK * IDX_MINOR

TC_BLOCK_SEQS = 64


def _sc_gather(idx_hbm, table_hbm, out_hbm, idx_v, rows_v, gsem, ssem0, ssem1):
    wid = lax.axis_index("s") * NC + lax.axis_index("c")
    idxr0 = wid * IDXR_PER_W
    row0 = wid * IDXR_PER_W * IDX_MINOR

    ssems = (ssem0, ssem1)
    store_handles = [None, None]
    for k in range(N_CHUNKS):
        p = k % 2
        if store_handles[p] is not None:
            store_handles[p].wait()
        pltpu.sync_copy(
            idx_hbm.at[pl.ds(idxr0 + k * IDXR_PER_CHUNK, IDXR_PER_CHUNK)],
            idx_v.at[p],
        )
        gathers = []
        for j in range(IDXR_PER_CHUNK):
            gathers.append(
                pltpu.async_copy(
                    table_hbm.at[idx_v.at[p, j]],
                    rows_v.at[p, pl.ds(j * IDX_MINOR, IDX_MINOR)],
                    gsem,
                )
            )
        for g in gathers:
            g.wait()
        store_handles[p] = pltpu.async_copy(
            rows_v.at[p],
            out_hbm.at[pl.ds(row0 + k * CHUNK, CHUNK)],
            ssems[p],
        )
    for h in store_handles:
        if h is not None:
            h.wait()


def _tc_add(rows_ref, pos_ref, out_ref):
    for b in range(TC_BLOCK_SEQS):
        out_ref[b] = (
            rows_ref[pl.ds(b * SEQ, SEQ), pl.ds(0, DIM)]
            + pos_ref[:, pl.ds(0, DIM)]
        )


def kernel(item, nbr_mask, i_ids, item_input_lookup, position_embedding):
    del nbr_mask, i_ids

    mesh = plsc.VectorSubcoreMesh(core_axis_name="c", subcore_axis_name="s")

    dense2d = jnp.pad(item_input_lookup, ((0, 0), (0, PDIM - DIM)))

    idx2d = item.reshape(IDX_ROWS, IDX_MINOR)
    gather = functools.partial(
        pl.kernel,
        mesh=mesh,
        out_type=jax.ShapeDtypeStruct((ROWS, PDIM), jnp.float32),
        scratch_types=[
            pltpu.VMEM((2, IDXR_PER_CHUNK, IDX_MINOR), jnp.int32),
            pltpu.VMEM((2, CHUNK, PDIM), jnp.float32),
            pltpu.SemaphoreType.DMA,
            pltpu.SemaphoreType.DMA,
            pltpu.SemaphoreType.DMA,
        ],
        compiler_params=pltpu.CompilerParams(use_tc_tiling_on_sc=False),
    )(_sc_gather)
    gathered = gather(idx2d, dense2d)

    pos_pad = jnp.pad(position_embedding.reshape(SEQ, DIM), ((0, 0), (0, PDIM - DIM)))
    out = pl.pallas_call(
        _tc_add,
        grid=(BATCH // TC_BLOCK_SEQS,),
        in_specs=[
            pl.BlockSpec((TC_BLOCK_SEQS * SEQ, PDIM), lambda i: (i, 0)),
            pl.BlockSpec((SEQ, PDIM), lambda i: (0, 0)),
        ],
        out_specs=pl.BlockSpec((TC_BLOCK_SEQS, SEQ, DIM), lambda i: (i, 0, 0)),
        out_shape=jax.ShapeDtypeStruct((BATCH, SEQ, DIM), jnp.float32),
    )(gathered, pos_pad)
    return out
